# baseline (speedup 1.0000x reference)
import jax
import jax.numpy as jnp
from jax.experimental import pallas as pl
from jax.experimental.pallas import tpu as pltpu

P = 8
N = 512
B = 64
EPS = 1e-06
NSEM = 8


def _body(starts_ref, ends_ref, dst_ref, s_ref, g_ref, out_ref, m_ref, sems):
    i = pl.program_id(0)

    s = s_ref[0]
    rows = jax.lax.broadcasted_iota(jnp.int32, (N, N), 0)
    cols = jax.lax.broadcasted_iota(jnp.int32, (N, N), 1)
    sz = jnp.where(rows == cols, 0.0, s)
    denom = jnp.clip(jnp.sum(jnp.abs(sz), axis=1, keepdims=True), EPS, None)
    graw = g_ref[0]
    g = jnp.maximum(graw, 0.0) + jnp.log1p(jnp.exp(-jnp.abs(graw))) + 1e-06
    gsum = jnp.clip(jnp.sum(g), EPS, None)
    scale = g * (N / gsum) / denom
    m_ref[i] = sz * scale

    def issue(k, c):
        pltpu.make_async_copy(
            m_ref.at[i], out_ref.at[dst_ref[k]], sems.at[jax.lax.rem(k, NSEM)]
        ).start()
        return c

    jax.lax.fori_loop(starts_ref[i], ends_ref[i], issue, 0)

    @pl.when(i == P - 1)
    def _():
        def w(k, c):
            pltpu.make_async_copy(
                m_ref.at[0], out_ref.at[0], sems.at[jax.lax.rem(k, NSEM)]
            ).wait()
            return c

        jax.lax.fori_loop(0, B, w, 0)


@jax.jit
def kernel(phases, S, G):
    phases = phases.astype(jnp.int32)
    order = jnp.argsort(phases)
    dst = order.astype(jnp.int32)
    counts = jnp.bincount(phases, length=P)
    ends = jnp.cumsum(counts).astype(jnp.int32)
    starts = (ends - counts).astype(jnp.int32)
    Gc = G.reshape(P, N, 1)

    grid_spec = pltpu.PrefetchScalarGridSpec(
        num_scalar_prefetch=3,
        grid=(P,),
        in_specs=[
            pl.BlockSpec((1, N, N), lambda i, st, en, d: (i, 0, 0)),
            pl.BlockSpec((1, N, 1), lambda i, st, en, d: (i, 0, 0)),
        ],
        out_specs=pl.BlockSpec(memory_space=pl.ANY),
        scratch_shapes=[
            pltpu.VMEM((P, N, N), jnp.float32),
            pltpu.SemaphoreType.DMA((NSEM,)),
        ],
    )

    out = pl.pallas_call(
        _body,
        grid_spec=grid_spec,
        out_shape=jax.ShapeDtypeStruct((B, N, N), jnp.float32),
    )(starts, ends, dst, S, Gc)
    return out


# speedup vs baseline: 1.7305x; 1.7305x over previous
import jax
import jax.numpy as jnp
from jax.experimental import pallas as pl
from jax.experimental.pallas import tpu as pltpu

P = 8
N = 512
B = 64
EPS = 1e-06


def _body(ph_ref, s_ref, g_ref, out_ref, m_ref, sem):
    i = pl.program_id(0)

    s = s_ref[0]
    rows = jax.lax.broadcasted_iota(jnp.int32, (N, N), 0)
    cols = jax.lax.broadcasted_iota(jnp.int32, (N, N), 1)
    sz = jnp.where(rows == cols, 0.0, s)
    denom = jnp.clip(jnp.sum(jnp.abs(sz), axis=1, keepdims=True), EPS, None)
    graw = g_ref[0]
    g = jnp.maximum(graw, 0.0) + jnp.log1p(jnp.exp(-jnp.abs(graw))) + 1e-06
    gsum = jnp.clip(jnp.sum(g), EPS, None)
    scale = g * (N / gsum) / denom
    m_ref[i] = sz * scale

    def issue(b, c):
        @pl.when(ph_ref[b] == i)
        def _():
            pltpu.make_async_copy(m_ref.at[i], out_ref.at[b], sem).start()

        return c

    jax.lax.fori_loop(0, B, issue, 0)

    @pl.when(i == P - 1)
    def _():
        def w(k, c):
            pltpu.make_async_copy(m_ref.at[0], out_ref.at[0], sem).wait()
            return c

        jax.lax.fori_loop(0, B, w, 0)


@jax.jit
def kernel(phases, S, G):
    phases = phases.astype(jnp.int32)
    Gc = G.reshape(P, N, 1)

    grid_spec = pltpu.PrefetchScalarGridSpec(
        num_scalar_prefetch=1,
        grid=(P,),
        in_specs=[
            pl.BlockSpec((1, N, N), lambda i, ph: (i, 0, 0)),
            pl.BlockSpec((1, N, 1), lambda i, ph: (i, 0, 0)),
        ],
        out_specs=pl.BlockSpec(memory_space=pl.ANY),
        scratch_shapes=[
            pltpu.VMEM((P, N, N), jnp.float32),
            pltpu.SemaphoreType.DMA,
        ],
    )

    out = pl.pallas_call(
        _body,
        grid_spec=grid_spec,
        out_shape=jax.ShapeDtypeStruct((B, N, N), jnp.float32),
    )(phases, S, Gc)
    return out
